# baseline (speedup 1.0000x reference)
import functools

import jax
import jax.numpy as jnp
from jax import lax
from jax.experimental import pallas as pl
from jax.experimental.pallas import tpu as pltpu

NUM_EXPERTS_ = 64
TOKENS_ = 128
HIDDEN_ = 1024
INTER_ = 512
TILE_ = 32


def _moe_kernel(x_ref, idx_ref, w_ref, up_ref, down_ref, out_ref):
    e = pl.program_id(0)

    @pl.when(e == 0)
    def _init():
        out_ref[...] = jnp.zeros_like(out_ref)

    idx = idx_ref[...]
    w = w_ref[...]
    match = idx == e
    c = jnp.sum(jnp.where(match, w, 0.0), axis=1, keepdims=True)
    m = jnp.any(match, axis=1, keepdims=True)
    m_f = m.astype(jnp.float32)

    t_iota = lax.broadcasted_iota(jnp.int32, (TOKENS_, TOKENS_), 0)
    j_iota = lax.broadcasted_iota(jnp.int32, (TOKENS_, TOKENS_), 1)
    ltri = (j_iota <= t_iota).astype(jnp.float32)
    csel = lax.dot(ltri, m_f, precision=lax.Precision.HIGHEST)
    cuns = lax.dot(ltri, 1.0 - m_f, precision=lax.Precision.HIGHEST)
    n = jnp.sum(m_f)
    rank = jnp.where(m, csel - 1.0, n + cuns - 1.0)
    rank_i = rank.astype(jnp.int32)

    n_i = jnp.sum(m.astype(jnp.int32))
    trips = (n_i + TILE_ - 1) // TILE_

    x = x_ref[...]
    up = up_ref[0]
    down = down_ref[0]
    col = lax.broadcasted_iota(jnp.int32, (TOKENS_, TILE_), 1)

    def body(tau, carry):
        base = tau * TILE_
        sel = (rank_i == col + base).astype(jnp.float32)
        xt = lax.dot_general(sel, x, (((0,), (0,)), ((), ())))
        wt = lax.dot_general(sel, c, (((0,), (0,)), ((), ())))
        h = lax.dot_general(xt, up, (((1,), (1,)), ((), ())))
        h = jnp.maximum(h, 0.0)
        y = lax.dot_general(h, down, (((1,), (1,)), ((), ())))
        y = y * wt
        out_ref[...] += lax.dot(sel, y)
        return carry

    lax.fori_loop(0, trips, body, 0)


@jax.jit
def kernel(hidden_states, top_k_index, top_k_weights, up_proj, down_proj):
    idx = top_k_index.astype(jnp.int32)
    grid = (NUM_EXPERTS_,)
    out = pl.pallas_call(
        _moe_kernel,
        grid=grid,
        in_specs=[
            pl.BlockSpec((TOKENS_, HIDDEN_), lambda e: (0, 0)),
            pl.BlockSpec((TOKENS_, 8), lambda e: (0, 0)),
            pl.BlockSpec((TOKENS_, 8), lambda e: (0, 0)),
            pl.BlockSpec((1, INTER_, HIDDEN_), lambda e: (e, 0, 0)),
            pl.BlockSpec((1, HIDDEN_, INTER_), lambda e: (e, 0, 0)),
        ],
        out_specs=pl.BlockSpec((TOKENS_, HIDDEN_), lambda e: (0, 0)),
        out_shape=jax.ShapeDtypeStruct((TOKENS_, HIDDEN_), jnp.float32),
        compiler_params=pltpu.CompilerParams(
            dimension_semantics=("arbitrary",),
        ),
    )(hidden_states, idx, top_k_weights, up_proj, down_proj)
    return out.astype(hidden_states.dtype)


# speedup vs baseline: 1.0273x; 1.0273x over previous
import jax
import jax.numpy as jnp
from jax import lax
from jax.experimental import pallas as pl
from jax.experimental.pallas import tpu as pltpu

NUM_EXPERTS_ = 64
TOKENS_ = 128
HIDDEN_ = 1024
INTER_ = 512
TILE_ = 32


def _moe_kernel(x_ref, idx_ref, w_ref, up_ref, down_ref, out_ref):
    e = pl.program_id(0)

    @pl.when(e == 0)
    def _init():
        out_ref[...] = jnp.zeros_like(out_ref)

    idx = idx_ref[...]
    w = w_ref[...]
    match = idx == e
    c = jnp.sum(jnp.where(match, w, 0.0), axis=1, keepdims=True)
    m = jnp.any(match, axis=1, keepdims=True)
    m_bf = m.astype(jnp.bfloat16)

    t_iota = lax.broadcasted_iota(jnp.int32, (TOKENS_, TOKENS_), 0)
    j_iota = lax.broadcasted_iota(jnp.int32, (TOKENS_, TOKENS_), 1)
    ltri = (j_iota <= t_iota).astype(jnp.bfloat16)
    csel = lax.dot(ltri, m_bf, preferred_element_type=jnp.float32)
    n = csel[TOKENS_ - 1, 0]
    row1 = (t_iota[:, :1] + 1).astype(jnp.float32)
    rank_bf = jnp.where(m, csel - 1.0, n + row1 - csel - 1.0).astype(jnp.bfloat16)

    n_i = n.astype(jnp.int32)
    trips = (n_i + TILE_ - 1) // TILE_

    x = x_ref[...].astype(jnp.bfloat16)
    up = up_ref[0].astype(jnp.bfloat16)
    down = down_ref[0].astype(jnp.bfloat16)
    c_bf = c.astype(jnp.bfloat16)
    col = lax.broadcasted_iota(jnp.int32, (TOKENS_, TILE_), 1).astype(
        jnp.bfloat16)

    def body(tau, carry):
        base = (tau * TILE_).astype(jnp.bfloat16)
        onehot = rank_bf == col + base
        sel = onehot.astype(jnp.bfloat16)
        selw = jnp.where(onehot, c_bf, jnp.bfloat16(0.0))
        xt = lax.dot_general(selw, x, (((0,), (0,)), ((), ())),
                             preferred_element_type=jnp.float32)
        h = lax.dot_general(xt.astype(jnp.bfloat16), up,
                            (((1,), (1,)), ((), ())),
                            preferred_element_type=jnp.float32)
        h = jnp.maximum(h, 0.0).astype(jnp.bfloat16)
        y = lax.dot_general(h, down, (((1,), (1,)), ((), ())),
                            preferred_element_type=jnp.float32)
        out_ref[...] += lax.dot(sel, y.astype(jnp.bfloat16),
                                preferred_element_type=jnp.float32)
        return carry

    lax.fori_loop(0, trips, body, 0)


@jax.jit
def kernel(hidden_states, top_k_index, top_k_weights, up_proj, down_proj):
    idx = top_k_index.astype(jnp.int32)
    out = pl.pallas_call(
        _moe_kernel,
        grid=(NUM_EXPERTS_,),
        in_specs=[
            pl.BlockSpec((TOKENS_, HIDDEN_), lambda e: (0, 0)),
            pl.BlockSpec((TOKENS_, 8), lambda e: (0, 0)),
            pl.BlockSpec((TOKENS_, 8), lambda e: (0, 0)),
            pl.BlockSpec((1, INTER_, HIDDEN_), lambda e: (e, 0, 0)),
            pl.BlockSpec((1, HIDDEN_, INTER_), lambda e: (e, 0, 0)),
        ],
        out_specs=pl.BlockSpec((TOKENS_, HIDDEN_), lambda e: (0, 0)),
        out_shape=jax.ShapeDtypeStruct((TOKENS_, HIDDEN_), jnp.float32),
        compiler_params=pltpu.CompilerParams(
            dimension_semantics=("arbitrary",),
        ),
    )(hidden_states, idx, top_k_weights, up_proj, down_proj)
    return out.astype(hidden_states.dtype)
